# Initial kernel scaffold; baseline (speedup 1.0000x reference)
import jax
import jax.numpy as jnp
from jax.experimental import pallas as pl


def kernel(entity_emb, edge_index, edge_type, weight, W1_w, W1_b, W2_w, W2_b, mess_dropout):
    raise NotImplementedError("write your pallas kernel here")


# speedup vs baseline: 2.0712x; 2.0712x over previous
import functools

import jax
import jax.numpy as jnp
from jax import lax
from jax.experimental import pallas as pl
from jax.experimental.pallas import tpu as pltpu
from jax.experimental.pallas import tpu_sc as plsc

N = 10000
NP = 10240
C = 128
E = 320000
NC = 2
NS = 16
NW = NC * NS
EPW = E // NW
K = 80
NCHUNK = EPW // K
RPW = NP // NS
L = 16

_mesh = plsc.VectorSubcoreMesh(core_axis_name="c", subcore_axis_name="s")


@functools.partial(
    pl.kernel,
    out_type=jax.ShapeDtypeStruct((NC, NP, C), jnp.float32),
    mesh=_mesh,
    scratch_types=(
        pltpu.VMEM((K,), jnp.int32),
        pltpu.VMEM((K,), jnp.int32),
        pltpu.VMEM((K,), jnp.int32),
        pltpu.VMEM((K, C), jnp.float32),
        pltpu.VMEM((K, C), jnp.float32),
        pltpu.VMEM_SHARED((NP, C), jnp.float32),
        pltpu.SemaphoreType.DMA,
        pltpu.SemaphoreType.DMA,
    ),
)
def _sc_scatter(ent_hbm, tail_hbm, type_hbm, head_hbm, w_hbm,
                sums_out,
                tail_v, type_v, head_v, ent_b, w_b,
                acc_s, sem0, sem1):
    cid = lax.axis_index("c")
    sid = lax.axis_index("s")
    wid = cid * NS + sid

    zero16 = jnp.zeros((L,), jnp.float32)

    def _fill_zero(r, _):
        for g in range(C // L):
            ent_b[r, pl.ds(g * L, L)] = zero16
        return 0
    lax.fori_loop(0, K, _fill_zero, 0)

    row0 = sid * RPW
    for k in range(RPW // K):
        pltpu.sync_copy(ent_b, acc_s.at[pl.ds(row0 + k * K, K)])
    plsc.subcore_barrier()

    ebase = wid * EPW

    def _chunk(j, _):
        off = ebase + j * K
        pltpu.sync_copy(tail_hbm.at[pl.ds(off, K)], tail_v)
        pltpu.sync_copy(type_hbm.at[pl.ds(off, K)], type_v)
        pltpu.sync_copy(head_hbm.at[pl.ds(off, K)], head_v)
        cp0 = pltpu.async_copy(ent_hbm.at[tail_v], ent_b, sem0)
        cp1 = pltpu.async_copy(w_hbm.at[type_v], w_b, sem1)
        cp0.wait()
        cp1.wait()

        def _mul(r, _):
            for g in range(C // L):
                s = pl.ds(g * L, L)
                ent_b[r, s] = ent_b[r, s] * w_b[r, s]
            return 0
        lax.fori_loop(0, K, _mul, 0)

        pltpu.sync_copy(ent_b, acc_s.at[head_v], add=True)
        return 0
    lax.fori_loop(0, NCHUNK, _chunk, 0)

    plsc.subcore_barrier()

    for k in range(RPW // K):
        r = row0 + k * K
        pltpu.sync_copy(acc_s.at[pl.ds(r, K)], ent_b)
        pltpu.sync_copy(ent_b, sums_out.at[cid, pl.ds(r, K)])


@functools.partial(
    pl.kernel,
    out_type=jax.ShapeDtypeStruct((NC, NP, C), jnp.float32),
    mesh=_mesh,
    scratch_types=(
        pltpu.VMEM((K,), jnp.int32),
        pltpu.VMEM((K, C), jnp.float32),
        pltpu.VMEM_SHARED((NP, C), jnp.float32),
    ),
)
def _sc_count(head_hbm, cnt_out, head_v, ones_b, acc_c):
    cid = lax.axis_index("c")
    sid = lax.axis_index("s")
    wid = cid * NS + sid

    zero16 = jnp.zeros((L,), jnp.float32)
    one16 = jnp.ones((L,), jnp.float32)

    def _fill(val):
        def _f(r, _):
            for g in range(C // L):
                ones_b[r, pl.ds(g * L, L)] = val
            return 0
        lax.fori_loop(0, K, _f, 0)

    _fill(zero16)
    row0 = sid * RPW
    for k in range(RPW // K):
        pltpu.sync_copy(ones_b, acc_c.at[pl.ds(row0 + k * K, K)])
    plsc.subcore_barrier()
    _fill(one16)

    ebase = wid * EPW

    def _chunk(j, _):
        off = ebase + j * K
        pltpu.sync_copy(head_hbm.at[pl.ds(off, K)], head_v)
        pltpu.sync_copy(ones_b, acc_c.at[head_v], add=True)
        return 0
    lax.fori_loop(0, NCHUNK, _chunk, 0)

    plsc.subcore_barrier()

    for k in range(RPW // K):
        r = row0 + k * K
        pltpu.sync_copy(acc_c.at[pl.ds(r, K)], ones_b)
        pltpu.sync_copy(ones_b, cnt_out.at[cid, pl.ds(r, K)])


def _leaky(x):
    return jnp.where(x >= 0, x, 0.01 * x)


def _tc_body(x_ref, sp_ref, cn_ref, w1_ref, b1_ref, w2_ref, b2_ref, out_ref):
    p = sp_ref[0] + sp_ref[1]
    cnt = cn_ref[0, :, 0:1] + cn_ref[1, :, 0:1]
    agg = p / jnp.maximum(cnt, 1.0)
    nrm = jnp.sqrt(jnp.sum(agg * agg, axis=1, keepdims=True))
    agg = agg / jnp.maximum(nrm, 1e-12)
    x = x_ref[...]
    dn = (((1,), (1,)), ((), ()))
    h1 = lax.dot_general(x + agg, w1_ref[...], dn,
                         precision=lax.Precision.HIGHEST,
                         preferred_element_type=jnp.float32)
    e1 = _leaky(h1 + b1_ref[...])
    cat = jnp.concatenate([x, agg], axis=1)
    h2 = lax.dot_general(cat, w2_ref[...], dn,
                         precision=lax.Precision.HIGHEST,
                         preferred_element_type=jnp.float32)
    e2 = _leaky(h2 + b2_ref[...])
    out_ref[...] = e1 + e2


_BT = 2048


@jax.jit
def _tc_dense(x, sums, cnts, w1, b1, w2, b2):
    grid = (NP // _BT,)
    return pl.pallas_call(
        _tc_body,
        grid=grid,
        in_specs=[
            pl.BlockSpec((_BT, C), lambda i: (i, 0)),
            pl.BlockSpec((NC, _BT, C), lambda i: (0, i, 0)),
            pl.BlockSpec((NC, _BT, C), lambda i: (0, i, 0)),
            pl.BlockSpec((C, C), lambda i: (0, 0)),
            pl.BlockSpec((1, C), lambda i: (0, 0)),
            pl.BlockSpec((C, 2 * C), lambda i: (0, 0)),
            pl.BlockSpec((1, C), lambda i: (0, 0)),
        ],
        out_specs=pl.BlockSpec((_BT, C), lambda i: (i, 0)),
        out_shape=jax.ShapeDtypeStruct((NP, C), jnp.float32),
    )(x, sums, cnts, w1, b1, w2, b2)


def kernel(entity_emb, edge_index, edge_type, weight, W1_w, W1_b, W2_w, W2_b,
           mess_dropout=False):
    head = edge_index[0]
    tail = edge_index[1]
    x = jnp.pad(entity_emb, ((0, NP - N), (0, 0)))
    cnts = _sc_count(head)
    for i in range(3):
        sums = _sc_scatter(x, tail, edge_type, head, weight)
        x = _tc_dense(x, sums, cnts, W1_w[i], W1_b[i].reshape(1, C),
                      W2_w[i], W2_b[i].reshape(1, C))
    return x[:N], weight
